# baseline (speedup 1.0000x reference)
import functools

import jax
import jax.numpy as jnp
from jax import lax
from jax.experimental import pallas as pl
from jax.experimental.pallas import tpu as pltpu
from jax.experimental.pallas import tpu_sc as plsc

N = 10000
E = 320000
D = 128
DE = 16
O = 128

NC = 2
NS = 16
NW = NC * NS

B = 80
NCHUNKS = E // B
ROWS_PER_TILE = 624
ROWS_REM = N - NS * ROWS_PER_TILE


def _p_body(x_ref, w_ref, b_ref, o_ref):
    o_ref[...] = (
        jnp.dot(x_ref[...], w_ref[...], preferred_element_type=jnp.float32)
        + b_ref[...]
    )


def _compute_p(x, w1ax, b1a):
    blk = 2000
    return pl.pallas_call(
        _p_body,
        grid=(N // blk,),
        in_specs=[
            pl.BlockSpec((blk, D), lambda i: (i, 0)),
            pl.BlockSpec((D, O), lambda i: (0, 0)),
            pl.BlockSpec((1, O), lambda i: (0, 0)),
        ],
        out_specs=pl.BlockSpec((blk, O), lambda i: (i, 0)),
        out_shape=jax.ShapeDtypeStruct((N, O), jnp.float32),
    )(x, w1ax, b1a.reshape(1, O))


def _q_body(ea_ref, w_ref, o_ref):
    o_ref[...] = jnp.dot(ea_ref[...], w_ref[...],
                         preferred_element_type=jnp.float32)


def _compute_q(ea, w1ae):
    blk = 3200
    return pl.pallas_call(
        _q_body,
        grid=(E // blk,),
        in_specs=[
            pl.BlockSpec((blk, DE), lambda i: (i, 0)),
            pl.BlockSpec((DE, O), lambda i: (0, 0)),
        ],
        out_specs=pl.BlockSpec((blk, O), lambda i: (i, 0)),
        out_shape=jax.ShapeDtypeStruct((E, O), jnp.float32),
    )(ea, w1ae)


NCH = NCHUNKS // NW


def _sc_body(p_hbm, q_hbm, col_hbm, row_hbm, z128_hbm,
             outs_hbm, outc_hbm,
             colv0, colv1, rowv0, rowv1, gv0, gv1, qv0, qv1, s_sh,
             sin0, sin1, sg0, sg1, ss0, ss1):
    c = lax.axis_index("c")
    s = lax.axis_index("s")
    w = c * NS + s
    base = s * ROWS_PER_TILE
    rbase = NS * ROWS_PER_TILE
    colv = (colv0, colv1)
    rowv = (rowv0, rowv1)
    gv = (gv0, gv1)
    qv = (qv0, qv1)
    sin = (sin0, sin1)
    sg = (sg0, sg1)
    ss = (ss0, ss1)

    def _zero_accum():
        pltpu.sync_copy(z128_hbm.at[pl.ds(0, ROWS_PER_TILE)],
                        s_sh.at[pl.ds(base, ROWS_PER_TILE)])

        @pl.when(s == NS - 1)
        def _init_rem():
            pltpu.sync_copy(z128_hbm.at[pl.ds(0, ROWS_REM)],
                            s_sh.at[pl.ds(rbase, ROWS_REM)])

    def _drain_accum(dst_hbm):
        pltpu.sync_copy(s_sh.at[pl.ds(base, ROWS_PER_TILE)],
                        dst_hbm.at[c, pl.ds(base, ROWS_PER_TILE)])

        @pl.when(s == NS - 1)
        def _drain_rem():
            pltpu.sync_copy(s_sh.at[pl.ds(rbase, ROWS_REM)],
                            dst_hbm.at[c, pl.ds(rbase, ROWS_REM)])

    def _off(k):
        return (w + jnp.minimum(k, NCH - 1) * NW) * B

    def _issue_inputs(k, b):
        off = _off(k)
        pltpu.async_copy(col_hbm.at[pl.ds(off, B)], colv[b], sin[b])
        pltpu.async_copy(row_hbm.at[pl.ds(off, B)], rowv[b], sin[b])
        pltpu.async_copy(q_hbm.at[pl.ds(off, B)], qv[b], sin[b])

    def _drain_inputs(k, b):
        off = _off(k)
        pltpu.make_async_copy(col_hbm.at[pl.ds(off, B)], colv[b], sin[b]).wait()
        pltpu.make_async_copy(row_hbm.at[pl.ds(off, B)], rowv[b], sin[b]).wait()
        pltpu.make_async_copy(q_hbm.at[pl.ds(off, B)], qv[b], sin[b]).wait()

    def _drain_scatter(b):
        pltpu.make_async_copy(gv[b], s_sh.at[rowv[b]], ss[b]).wait()

    def _ew(b):
        @plsc.parallel_loop(0, B, unroll=2)
        def _row(r):
            for kk in range(O // 16):
                sl = pl.ds(kk * 16, 16)
                gv[b][r, sl] = jnp.maximum(gv[b][r, sl] + qv[b][r, sl], 0.0)

    def _step(k, b, drain_prev_scatter=True):
        nb = 1 - b
        _drain_inputs(k, b)
        gcp = pltpu.async_copy(p_hbm.at[colv[b]], gv[b], sg[b])
        if drain_prev_scatter:
            _drain_scatter(nb)
        _issue_inputs(k + 1, nb)
        gcp.wait()
        _ew(b)
        pltpu.async_copy(gv[b], s_sh.at[rowv[b]], ss[b], add=True)

    _zero_accum()
    plsc.subcore_barrier()

    _issue_inputs(0, 0)
    _step(0, 0, drain_prev_scatter=False)

    @pl.loop(0, (NCH - 1) // 2)
    def _chunk(g):
        _step(1 + 2 * g, 1)
        _step(2 + 2 * g, 0)

    _drain_scatter(0)
    _drain_inputs(NCH, 1)

    plsc.subcore_barrier()
    _drain_accum(outs_hbm)
    plsc.subcore_barrier()

    _zero_accum()
    ones16 = jnp.ones((16,), jnp.float32)

    @pl.loop(0, B)
    def _fill_ones(r):
        for kk in range(O // 16):
            gv0[r, pl.ds(kk * 16, 16)] = ones16

    plsc.subcore_barrier()

    def _issue_row(k, b):
        pltpu.async_copy(row_hbm.at[pl.ds(_off(k), B)], rowv[b], sin[b])

    def _drain_row(k, b):
        pltpu.make_async_copy(row_hbm.at[pl.ds(_off(k), B)], rowv[b],
                              sin[b]).wait()

    def _drain_cnt_scatter(b):
        pltpu.make_async_copy(gv0, s_sh.at[rowv[b]], ss[b]).wait()

    def _cnt_step(k, b, drain_prev_scatter=True):
        nb = 1 - b
        _drain_row(k, b)
        if drain_prev_scatter:
            _drain_cnt_scatter(nb)
        _issue_row(k + 1, nb)
        pltpu.async_copy(gv0, s_sh.at[rowv[b]], ss[b], add=True)

    _issue_row(0, 0)
    _cnt_step(0, 0, drain_prev_scatter=False)

    @pl.loop(0, (NCH - 1) // 2)
    def _chunk_cnt(g):
        _cnt_step(1 + 2 * g, 1)
        _cnt_step(2 + 2 * g, 0)

    _drain_cnt_scatter(0)
    _drain_row(NCH, 1)

    plsc.subcore_barrier()
    _drain_accum(outc_hbm)


_sc_scatter = functools.partial(
    pl.kernel,
    out_type=[
        jax.ShapeDtypeStruct((NC, N, O), jnp.float32),
        jax.ShapeDtypeStruct((NC, N, O), jnp.float32),
    ],
    mesh=plsc.VectorSubcoreMesh(
        core_axis_name="c", subcore_axis_name="s", num_cores=NC, num_subcores=NS
    ),
    scratch_types=[
        pltpu.VMEM((B,), jnp.int32),
        pltpu.VMEM((B,), jnp.int32),
        pltpu.VMEM((B,), jnp.int32),
        pltpu.VMEM((B,), jnp.int32),
        pltpu.VMEM((B, O), jnp.float32),
        pltpu.VMEM((B, O), jnp.float32),
        pltpu.VMEM((B, O), jnp.float32),
        pltpu.VMEM((B, O), jnp.float32),
        pltpu.VMEM_SHARED((N, O), jnp.float32),
        pltpu.SemaphoreType.DMA,
        pltpu.SemaphoreType.DMA,
        pltpu.SemaphoreType.DMA,
        pltpu.SemaphoreType.DMA,
        pltpu.SemaphoreType.DMA,
        pltpu.SemaphoreType.DMA,
    ],
)(_sc_body)


def _node_body(x_ref, sp_ref, cp_ref, w1b_ref, b1b_ref, w2ax_ref,
               w2am_ref, b2a_ref, w2b_ref, b2b_ref, wro_ref, wrx_ref, br_ref,
               o_ref):
    x = x_ref[...]
    ssum = sp_ref[0] + sp_ref[1]
    cnt = cp_ref[0, :, 0:1] + cp_ref[1, :, 0:1]
    m = ssum / jnp.maximum(cnt, 1.0)
    gate = (cnt > 0.0).astype(jnp.float32)
    mean = (
        jnp.dot(m, w1b_ref[...], preferred_element_type=jnp.float32)
        + b1b_ref[...] * gate
    )
    h = jnp.maximum(
        jnp.dot(x, w2ax_ref[...], preferred_element_type=jnp.float32)
        + jnp.dot(mean, w2am_ref[...], preferred_element_type=jnp.float32)
        + b2a_ref[...],
        0.0,
    )
    o2 = jnp.dot(h, w2b_ref[...], preferred_element_type=jnp.float32) + b2b_ref[...]
    o_ref[...] = (
        jnp.dot(o2, wro_ref[...], preferred_element_type=jnp.float32)
        + jnp.dot(x, wrx_ref[...], preferred_element_type=jnp.float32)
        + br_ref[...]
    )


def _node_update(x, s_partials, c_partials, w1b, b1b, w2ax, w2am, b2a, w2b, b2b,
                 wro, wrx, br):
    blk = 2000
    full = lambda shape: pl.BlockSpec(shape, lambda i: tuple(0 for _ in shape))
    return pl.pallas_call(
        _node_body,
        grid=(N // blk,),
        in_specs=[
            pl.BlockSpec((blk, D), lambda i: (i, 0)),
            pl.BlockSpec((NC, blk, O), lambda i: (0, i, 0)),
            pl.BlockSpec((NC, blk, O), lambda i: (0, i, 0)),
            full((O, O)),
            full((1, O)),
            full((D, O)),
            full((O, O)),
            full((1, O)),
            full((O, O)),
            full((1, O)),
            full((O, O)),
            full((D, O)),
            full((1, O)),
        ],
        out_specs=pl.BlockSpec((blk, O), lambda i: (i, 0)),
        out_shape=jax.ShapeDtypeStruct((N, O), jnp.float32),
    )(x, s_partials, c_partials, w1b, b1b.reshape(1, O), w2ax, w2am,
      b2a.reshape(1, O), w2b, b2b.reshape(1, O), wro, wrx, br.reshape(1, O))


def kernel(src_node_features, edge_index, edge_attr, u, batch,
           W1a, b1a, W1b, b1b, W2a, b2a, W2b, b2b, Wr, br):
    x = src_node_features
    row = edge_index[0]
    col = edge_index[1]

    p = _compute_p(x, W1a[:D], b1a)
    q = _compute_q(edge_attr, W1a[D:])

    z128 = jnp.zeros((ROWS_PER_TILE, O), dtype=jnp.float32)
    s_partials, c_partials = _sc_scatter(p, q, col, row, z128)

    return _node_update(x, s_partials, c_partials, W1b, b1b,
                        W2a[:D], W2a[D:], b2a, W2b, b2b, Wr[:O], Wr[O:], br)


# speedup vs baseline: 1.0131x; 1.0131x over previous
import functools

import jax
import jax.numpy as jnp
from jax import lax
from jax.experimental import pallas as pl
from jax.experimental.pallas import tpu as pltpu
from jax.experimental.pallas import tpu_sc as plsc

N = 10000
E = 320000
D = 128
DE = 16
O = 128

NC = 2
NS = 16
NW = NC * NS

B = 80
NCHUNKS = E // B
ROWS_PER_TILE = 624
ROWS_REM = N - NS * ROWS_PER_TILE


PBLK = 2000
NPB = N // PBLK


def _pq_body(ea_ref, we_ref, x_ref, wx_ref, b_ref, oq_ref, op_ref):
    oq_ref[...] = jnp.dot(ea_ref[...], we_ref[...],
                          preferred_element_type=jnp.float32)

    @pl.when(pl.program_id(0) < NPB)
    def _p():
        op_ref[...] = (
            jnp.dot(x_ref[...], wx_ref[...], preferred_element_type=jnp.float32)
            + b_ref[...]
        )


def _compute_pq(ea, w1ae, x, w1ax, b1a):
    blk = 3200
    return pl.pallas_call(
        _pq_body,
        grid=(E // blk,),
        in_specs=[
            pl.BlockSpec((blk, DE), lambda i: (i, 0)),
            pl.BlockSpec((DE, O), lambda i: (0, 0)),
            pl.BlockSpec((PBLK, D), lambda i: (jnp.minimum(i, NPB - 1), 0)),
            pl.BlockSpec((D, O), lambda i: (0, 0)),
            pl.BlockSpec((1, O), lambda i: (0, 0)),
        ],
        out_specs=[
            pl.BlockSpec((blk, O), lambda i: (i, 0)),
            pl.BlockSpec((PBLK, O), lambda i: (jnp.minimum(i, NPB - 1), 0)),
        ],
        out_shape=[
            jax.ShapeDtypeStruct((E, O), jnp.float32),
            jax.ShapeDtypeStruct((N, O), jnp.float32),
        ],
    )(ea, w1ae, x, w1ax, b1a.reshape(1, O))


NCH = NCHUNKS // NW


def _sc_body(p_hbm, q_hbm, col_hbm, row_hbm, z128_hbm,
             outs_hbm, outc_hbm,
             colv0, colv1, rowv0, rowv1, gv0, gv1, qv0, qv1, s_sh,
             sin0, sin1, sg0, sg1, ss0, ss1):
    c = lax.axis_index("c")
    s = lax.axis_index("s")
    w = c * NS + s
    base = s * ROWS_PER_TILE
    rbase = NS * ROWS_PER_TILE
    colv = (colv0, colv1)
    rowv = (rowv0, rowv1)
    gv = (gv0, gv1)
    qv = (qv0, qv1)
    sin = (sin0, sin1)
    sg = (sg0, sg1)
    ss = (ss0, ss1)

    def _zero_accum():
        pltpu.sync_copy(z128_hbm.at[pl.ds(0, ROWS_PER_TILE)],
                        s_sh.at[pl.ds(base, ROWS_PER_TILE)])

        @pl.when(s == NS - 1)
        def _init_rem():
            pltpu.sync_copy(z128_hbm.at[pl.ds(0, ROWS_REM)],
                            s_sh.at[pl.ds(rbase, ROWS_REM)])

    def _drain_accum(dst_hbm):
        pltpu.sync_copy(s_sh.at[pl.ds(base, ROWS_PER_TILE)],
                        dst_hbm.at[c, pl.ds(base, ROWS_PER_TILE)])

        @pl.when(s == NS - 1)
        def _drain_rem():
            pltpu.sync_copy(s_sh.at[pl.ds(rbase, ROWS_REM)],
                            dst_hbm.at[c, pl.ds(rbase, ROWS_REM)])

    def _off(k):
        return (w + jnp.minimum(k, NCH - 1) * NW) * B

    def _issue_inputs(k, b):
        off = _off(k)
        pltpu.async_copy(col_hbm.at[pl.ds(off, B)], colv[b], sin[b])
        pltpu.async_copy(row_hbm.at[pl.ds(off, B)], rowv[b], sin[b])
        pltpu.async_copy(q_hbm.at[pl.ds(off, B)], qv[b], sin[b])

    def _drain_inputs(k, b):
        off = _off(k)
        pltpu.make_async_copy(col_hbm.at[pl.ds(off, B)], colv[b], sin[b]).wait()
        pltpu.make_async_copy(row_hbm.at[pl.ds(off, B)], rowv[b], sin[b]).wait()
        pltpu.make_async_copy(q_hbm.at[pl.ds(off, B)], qv[b], sin[b]).wait()

    def _drain_scatter(b):
        pltpu.make_async_copy(gv[b], s_sh.at[rowv[b]], ss[b]).wait()

    def _ew(b):
        @pl.loop(0, B)
        def _row(r):
            for kk in range(O // 16):
                sl = pl.ds(kk * 16, 16)
                gv[b][r, sl] = jnp.maximum(gv[b][r, sl] + qv[b][r, sl], 0.0)

    def _step(k, b, drain_prev_scatter=True):
        nb = 1 - b
        _drain_inputs(k, b)
        gcp = pltpu.async_copy(p_hbm.at[colv[b]], gv[b], sg[b])
        if drain_prev_scatter:
            _drain_scatter(nb)
        _issue_inputs(k + 1, nb)
        gcp.wait()
        _ew(b)
        pltpu.async_copy(gv[b], s_sh.at[rowv[b]], ss[b], add=True)

    _zero_accum()
    plsc.subcore_barrier()

    _issue_inputs(0, 0)
    _step(0, 0, drain_prev_scatter=False)

    @pl.loop(0, (NCH - 1) // 2)
    def _chunk(g):
        _step(1 + 2 * g, 1)
        _step(2 + 2 * g, 0)

    _drain_scatter(0)
    _drain_inputs(NCH, 1)

    plsc.subcore_barrier()
    _drain_accum(outs_hbm)
    plsc.subcore_barrier()

    _zero_accum()
    ones16 = jnp.ones((16,), jnp.float32)

    @pl.loop(0, B)
    def _fill_ones(r):
        for kk in range(O // 16):
            gv0[r, pl.ds(kk * 16, 16)] = ones16

    plsc.subcore_barrier()

    def _issue_row(k, b):
        pltpu.async_copy(row_hbm.at[pl.ds(_off(k), B)], rowv[b], sin[b])

    def _drain_row(k, b):
        pltpu.make_async_copy(row_hbm.at[pl.ds(_off(k), B)], rowv[b],
                              sin[b]).wait()

    def _drain_cnt_scatter(b):
        pltpu.make_async_copy(gv0, s_sh.at[rowv[b]], ss[b]).wait()

    def _cnt_step(k, b, drain_prev_scatter=True):
        nb = 1 - b
        _drain_row(k, b)
        if drain_prev_scatter:
            _drain_cnt_scatter(nb)
        _issue_row(k + 1, nb)
        pltpu.async_copy(gv0, s_sh.at[rowv[b]], ss[b], add=True)

    _issue_row(0, 0)
    _cnt_step(0, 0, drain_prev_scatter=False)

    @pl.loop(0, (NCH - 1) // 2)
    def _chunk_cnt(g):
        _cnt_step(1 + 2 * g, 1)
        _cnt_step(2 + 2 * g, 0)

    _drain_cnt_scatter(0)
    _drain_row(NCH, 1)

    plsc.subcore_barrier()
    _drain_accum(outc_hbm)


_sc_scatter = functools.partial(
    pl.kernel,
    out_type=[
        jax.ShapeDtypeStruct((NC, N, O), jnp.float32),
        jax.ShapeDtypeStruct((NC, N, O), jnp.float32),
    ],
    mesh=plsc.VectorSubcoreMesh(
        core_axis_name="c", subcore_axis_name="s", num_cores=NC, num_subcores=NS
    ),
    scratch_types=[
        pltpu.VMEM((B,), jnp.int32),
        pltpu.VMEM((B,), jnp.int32),
        pltpu.VMEM((B,), jnp.int32),
        pltpu.VMEM((B,), jnp.int32),
        pltpu.VMEM((B, O), jnp.float32),
        pltpu.VMEM((B, O), jnp.float32),
        pltpu.VMEM((B, O), jnp.float32),
        pltpu.VMEM((B, O), jnp.float32),
        pltpu.VMEM_SHARED((N, O), jnp.float32),
        pltpu.SemaphoreType.DMA,
        pltpu.SemaphoreType.DMA,
        pltpu.SemaphoreType.DMA,
        pltpu.SemaphoreType.DMA,
        pltpu.SemaphoreType.DMA,
        pltpu.SemaphoreType.DMA,
    ],
)(_sc_body)


def _node_body(x_ref, sp_ref, cp_ref, w1b_ref, b1b_ref, w2ax_ref,
               w2am_ref, b2a_ref, w2b_ref, b2b_ref, wro_ref, wrx_ref, br_ref,
               o_ref):
    x = x_ref[...]
    ssum = sp_ref[0] + sp_ref[1]
    cnt = cp_ref[0, :, 0:1] + cp_ref[1, :, 0:1]
    m = ssum / jnp.maximum(cnt, 1.0)
    gate = (cnt > 0.0).astype(jnp.float32)
    mean = (
        jnp.dot(m, w1b_ref[...], preferred_element_type=jnp.float32)
        + b1b_ref[...] * gate
    )
    h = jnp.maximum(
        jnp.dot(x, w2ax_ref[...], preferred_element_type=jnp.float32)
        + jnp.dot(mean, w2am_ref[...], preferred_element_type=jnp.float32)
        + b2a_ref[...],
        0.0,
    )
    o2 = jnp.dot(h, w2b_ref[...], preferred_element_type=jnp.float32) + b2b_ref[...]
    o_ref[...] = (
        jnp.dot(o2, wro_ref[...], preferred_element_type=jnp.float32)
        + jnp.dot(x, wrx_ref[...], preferred_element_type=jnp.float32)
        + br_ref[...]
    )


def _node_update(x, s_partials, c_partials, w1b, b1b, w2ax, w2am, b2a, w2b, b2b,
                 wro, wrx, br):
    blk = 2000
    full = lambda shape: pl.BlockSpec(shape, lambda i: tuple(0 for _ in shape))
    return pl.pallas_call(
        _node_body,
        grid=(N // blk,),
        in_specs=[
            pl.BlockSpec((blk, D), lambda i: (i, 0)),
            pl.BlockSpec((NC, blk, O), lambda i: (0, i, 0)),
            pl.BlockSpec((NC, blk, O), lambda i: (0, i, 0)),
            full((O, O)),
            full((1, O)),
            full((D, O)),
            full((O, O)),
            full((1, O)),
            full((O, O)),
            full((1, O)),
            full((O, O)),
            full((D, O)),
            full((1, O)),
        ],
        out_specs=pl.BlockSpec((blk, O), lambda i: (i, 0)),
        out_shape=jax.ShapeDtypeStruct((N, O), jnp.float32),
    )(x, s_partials, c_partials, w1b, b1b.reshape(1, O), w2ax, w2am,
      b2a.reshape(1, O), w2b, b2b.reshape(1, O), wro, wrx, br.reshape(1, O))


def kernel(src_node_features, edge_index, edge_attr, u, batch,
           W1a, b1a, W1b, b1b, W2a, b2a, W2b, b2b, Wr, br):
    x = src_node_features
    row = edge_index[0]
    col = edge_index[1]

    q, p = _compute_pq(edge_attr, W1a[D:], x, W1a[:D], b1a)

    z128 = jnp.zeros((ROWS_PER_TILE, O), dtype=jnp.float32)
    s_partials, c_partials = _sc_scatter(p, q, col, row, z128)

    return _node_update(x, s_partials, c_partials, W1b, b1b,
                        W2a[:D], W2a[D:], b2a, W2b, b2b, Wr[:O], Wr[O:], br)
